# Initial kernel scaffold; baseline (speedup 1.0000x reference)
import jax
import jax.numpy as jnp
from jax.experimental import pallas as pl


def kernel(adj_matrix, emb, W1, b1, W2, b2, W3, b3, Wmu, bmu, Wlv, blv, P1, p1, P2, p2, P3, p3):
    raise NotImplementedError("write your pallas kernel here")


# speedup vs baseline: 61.4997x; 61.4997x over previous
import functools

import jax
import jax.numpy as jnp
from jax.experimental import pallas as pl

_TILE = 128


def _encoder_body(adj_ref, emb_ref, W1_ref, b1_ref, W2_ref, b2_ref, W3_ref,
                  b3_ref, Wmu_ref, bmu_ref, Wlv_ref, blv_ref, P1a_ref,
                  P1b_ref, p1_ref, mu_ref, lv_ref, u_ref, v_ref):
    f32 = jnp.float32
    hi = jax.lax.Precision.HIGHEST
    adj = adj_ref[...]
    n = adj.shape[0]
    ones = jnp.ones((n, 1), f32)
    deg = jax.lax.dot_general(adj, ones, (((0,), (0,)), ((), ())),
                              preferred_element_type=f32, precision=hi) + 1.0
    dinv = jnp.where(deg > 0, jax.lax.rsqrt(deg), 0.0)

    def gcn_layer(h, W_ref, b_ref):
        g = jnp.dot(h, W_ref[...], preferred_element_type=f32,
                    precision=hi) * dinv
        agg = jax.lax.dot_general(adj, g, (((0,), (0,)), ((), ())),
                                  preferred_element_type=f32, precision=hi)
        return (agg + g) * dinv + b_ref[...]

    h1 = jax.nn.relu(gcn_layer(emb_ref[...], W1_ref, b1_ref))
    h2 = jax.nn.relu(gcn_layer(h1, W2_ref, b2_ref)) + h1
    h3 = jax.nn.relu(gcn_layer(h2, W3_ref, b3_ref)) + h2
    mu = jnp.dot(h3, Wmu_ref[...], preferred_element_type=f32,
                 precision=hi) + bmu_ref[...]
    lv = jnp.dot(h3, Wlv_ref[...], preferred_element_type=f32,
                 precision=hi) + blv_ref[...]
    mu_ref[...] = mu
    lv_ref[...] = lv
    u_ref[...] = jnp.dot(mu, P1a_ref[...], preferred_element_type=f32,
                         precision=hi) + p1_ref[...]
    v_ref[...] = jnp.dot(mu, P1b_ref[...], preferred_element_type=f32,
                         precision=hi)


_CHUNK = 32


def _pair_mlp(a, b, P2, p2, P3r, p3):
    t = a.shape[0]
    outs = []
    for g in range(t // _CHUNK):
        ac = a[g * _CHUNK:(g + 1) * _CHUNK]
        hh = jax.nn.relu(ac[:, None, :] + b[None, :, :])
        hh = hh.reshape(_CHUNK * t, -1)
        h2 = jax.nn.relu(jnp.dot(hh, P2, preferred_element_type=jnp.float32,
                                 precision=jax.lax.Precision.HIGHEST) + p2)
        logit = jnp.sum(h2.reshape(_CHUNK, t, -1) * P3r[None, :, :],
                        axis=-1) + p3
        outs.append(logit)
    return jax.nn.sigmoid(jnp.concatenate(outs, axis=0))


def _decoder_body(ur_ref, vr_ref, uc_ref, vc_ref, P2_ref, p2_ref, P3r_ref,
                  p3_ref, out_ref):
    bi = pl.program_id(0)
    bj = pl.program_id(1)
    upper = bi < bj
    ur, vr = ur_ref[...], vr_ref[...]
    uc, vc = uc_ref[...], vc_ref[...]
    P2, p2 = P2_ref[...], p2_ref[...]
    P3r, p3 = P3r_ref[...], p3_ref[0, 0]
    a = jnp.where(upper, ur, vr)
    b = jnp.where(upper, vc, uc)
    s = _pair_mlp(a, b, P2, p2, P3r, p3)

    @pl.when(bi != bj)
    def _():
        out_ref[...] = s

    @pl.when(bi == bj)
    def _():
        s_up = _pair_mlp(ur, vc, P2, p2, P3r, p3)
        t = s.shape[0]
        r = jax.lax.broadcasted_iota(jnp.int32, (t, t), 0)
        c = jax.lax.broadcasted_iota(jnp.int32, (t, t), 1)
        out_ref[...] = jnp.where(r < c, s_up, jnp.where(r > c, s, 0.0))


def kernel(adj_matrix, emb, W1, b1, W2, b2, W3, b3, Wmu, bmu, Wlv, blv,
           P1, p1, P2, p2, P3, p3):
    n = adj_matrix.shape[0]
    L = Wmu.shape[1]
    f32 = jnp.float32

    mu, logvar, U, V = pl.pallas_call(
        _encoder_body,
        out_shape=(
            jax.ShapeDtypeStruct((n, L), f32),
            jax.ShapeDtypeStruct((n, L), f32),
            jax.ShapeDtypeStruct((n, P1.shape[1]), f32),
            jax.ShapeDtypeStruct((n, P1.shape[1]), f32),
        ),
    )(adj_matrix, emb, W1, b1.reshape(1, -1), W2, b2.reshape(1, -1), W3,
      b3.reshape(1, -1), Wmu, bmu.reshape(1, -1), Wlv, blv.reshape(1, -1),
      P1[:L], P1[L:], p1.reshape(1, -1))

    grid = (n // _TILE, n // _TILE)
    row_spec = pl.BlockSpec((_TILE, P1.shape[1]), lambda i, j: (i, 0))
    col_spec = pl.BlockSpec((_TILE, P1.shape[1]), lambda i, j: (j, 0))
    full = lambda shape: pl.BlockSpec(shape, lambda i, j: (0,) * len(shape))
    adj_pred = pl.pallas_call(
        _decoder_body,
        grid=grid,
        in_specs=[row_spec, row_spec, col_spec, col_spec,
                  full(P2.shape), full((1, P2.shape[1])),
                  full((1, P2.shape[1])), full((1, 1))],
        out_specs=pl.BlockSpec((_TILE, _TILE), lambda i, j: (i, j)),
        out_shape=jax.ShapeDtypeStruct((n, n), f32),
    )(U, V, U, V, P2, p2.reshape(1, -1), P3.reshape(1, -1),
      p3.reshape(1, 1))

    return (adj_pred, mu, logvar, mu)


# speedup vs baseline: 109.2744x; 1.7768x over previous
import jax
import jax.numpy as jnp
from jax.experimental import pallas as pl

_TILE = 128
_G = 8


def _encoder_body(adj_ref, emb_ref, W1_ref, b1_ref, W2_ref, b2_ref, W3_ref,
                  b3_ref, Wmu_ref, bmu_ref, Wlv_ref, blv_ref, P1aT_ref,
                  P1bT_ref, p1c_ref, mu_ref, lv_ref, ut_ref, vt_ref):
    f32 = jnp.float32
    hi = jax.lax.Precision.HIGHEST
    adj = adj_ref[...]
    n = adj.shape[0]
    ones = jnp.ones((n, 1), f32)
    deg = jax.lax.dot_general(adj, ones, (((0,), (0,)), ((), ())),
                              preferred_element_type=f32, precision=hi) + 1.0
    dinv = jnp.where(deg > 0, jax.lax.rsqrt(deg), 0.0)

    def gcn_layer(h, W_ref, b_ref):
        g = jnp.dot(h, W_ref[...], preferred_element_type=f32,
                    precision=hi) * dinv
        agg = jax.lax.dot_general(adj, g, (((0,), (0,)), ((), ())),
                                  preferred_element_type=f32, precision=hi)
        return (agg + g) * dinv + b_ref[...]

    h1 = jax.nn.relu(gcn_layer(emb_ref[...], W1_ref, b1_ref))
    h2 = jax.nn.relu(gcn_layer(h1, W2_ref, b2_ref)) + h1
    h3 = jax.nn.relu(gcn_layer(h2, W3_ref, b3_ref)) + h2
    mu = jnp.dot(h3, Wmu_ref[...], preferred_element_type=f32,
                 precision=hi) + bmu_ref[...]
    lv = jnp.dot(h3, Wlv_ref[...], preferred_element_type=f32,
                 precision=hi) + blv_ref[...]
    mu_ref[...] = mu
    lv_ref[...] = lv
    muT = mu.T
    ut_ref[...] = jnp.dot(P1aT_ref[...], muT, preferred_element_type=f32,
                          precision=hi) + p1c_ref[...]
    vt_ref[...] = jnp.dot(P1bT_ref[...], muT, preferred_element_type=f32,
                          precision=hi)


def _chunk_rows(g, acol, bblk, P2T, p2c, P3r, p3):
    t = bblk.shape[1]
    hi = jax.lax.Precision.HIGHEST
    cols = [acol[:, g * _G + r][:, None] + bblk for r in range(_G)]
    hh = jax.nn.relu(jnp.concatenate(cols, axis=1))
    h2 = jax.nn.relu(jnp.dot(P2T, hh, preferred_element_type=jnp.float32,
                             precision=hi) + p2c)
    logit = jnp.dot(P3r, h2, preferred_element_type=jnp.float32,
                    precision=hi) + p3
    return jax.nn.sigmoid(logit).reshape(_G, t)


def _decoder_body(ur_ref, vr_ref, uc_ref, vc_ref, P2T_ref, p2c_ref, P3r_ref,
                  p3_ref, out_ref):
    bi = pl.program_id(0)
    bj = pl.program_id(1)
    ur, vr = ur_ref[...], vr_ref[...]
    uc, vc = uc_ref[...], vc_ref[...]
    P2T, p2c = P2T_ref[...], p2c_ref[...]
    P3r, p3 = P3r_ref[...], p3_ref[0, 0]
    t = out_ref.shape[0]

    @pl.when(bi != bj)
    def _():
        upper = bi < bj
        acol = jnp.where(upper, ur, vr)
        bblk = jnp.where(upper, vc, uc)
        for g in range(t // _G):
            out_ref[pl.ds(g * _G, _G), :] = _chunk_rows(
                g, acol, bblk, P2T, p2c, P3r, p3)

    @pl.when(bi == bj)
    def _():
        for g in range(t // _G):
            s_up = _chunk_rows(g, ur, vc, P2T, p2c, P3r, p3)
            s_lo = _chunk_rows(g, vr, uc, P2T, p2c, P3r, p3)
            r = g * _G + jax.lax.broadcasted_iota(jnp.int32, (_G, t), 0)
            c = jax.lax.broadcasted_iota(jnp.int32, (_G, t), 1)
            out_ref[pl.ds(g * _G, _G), :] = jnp.where(
                r < c, s_up, jnp.where(r > c, s_lo, 0.0))


def kernel(adj_matrix, emb, W1, b1, W2, b2, W3, b3, Wmu, bmu, Wlv, blv,
           P1, p1, P2, p2, P3, p3):
    n = adj_matrix.shape[0]
    L = Wmu.shape[1]
    H2 = P1.shape[1]
    f32 = jnp.float32

    mu, logvar, UT, VT = pl.pallas_call(
        _encoder_body,
        out_shape=(
            jax.ShapeDtypeStruct((n, L), f32),
            jax.ShapeDtypeStruct((n, L), f32),
            jax.ShapeDtypeStruct((H2, n), f32),
            jax.ShapeDtypeStruct((H2, n), f32),
        ),
    )(adj_matrix, emb, W1, b1.reshape(1, -1), W2, b2.reshape(1, -1), W3,
      b3.reshape(1, -1), Wmu, bmu.reshape(1, -1), Wlv, blv.reshape(1, -1),
      P1[:L].T, P1[L:].T, p1.reshape(-1, 1))

    grid = (n // _TILE, n // _TILE)
    row_spec = pl.BlockSpec((H2, _TILE), lambda i, j: (0, i))
    col_spec = pl.BlockSpec((H2, _TILE), lambda i, j: (0, j))
    full = lambda shape: pl.BlockSpec(shape, lambda i, j: (0,) * len(shape))
    adj_pred = pl.pallas_call(
        _decoder_body,
        grid=grid,
        in_specs=[row_spec, row_spec, col_spec, col_spec,
                  full((P2.shape[1], P2.shape[0])), full((P2.shape[1], 1)),
                  full((1, P2.shape[1])), full((1, 1))],
        out_specs=pl.BlockSpec((_TILE, _TILE), lambda i, j: (i, j)),
        out_shape=jax.ShapeDtypeStruct((n, n), f32),
    )(UT, VT, UT, VT, P2.T, p2.reshape(-1, 1), P3.reshape(1, -1),
      p3.reshape(1, 1))

    return (adj_pred, mu, logvar, mu)


# speedup vs baseline: 248.5675x; 2.2747x over previous
import jax
import jax.numpy as jnp
import numpy as np
from jax.experimental import pallas as pl
from jax.experimental.pallas import tpu as pltpu

_TILE = 128
_G = 8


def _encoder_body(adj_ref, emb_ref, W1_ref, b1_ref, W2_ref, b2_ref, W3_ref,
                  b3_ref, Wmu_ref, bmu_ref, Wlv_ref, blv_ref, P1aT_ref,
                  P1bT_ref, p1c_ref, mu_ref, lv_ref, ut_ref, vt_ref):
    f32 = jnp.float32
    bf16 = jnp.bfloat16
    hi = jax.lax.Precision.HIGHEST
    adj = adj_ref[...]
    n = adj.shape[0]
    adjb = adj.astype(bf16)
    ones = jnp.ones((n, 1), bf16)
    deg = jax.lax.dot_general(adjb, ones, (((0,), (0,)), ((), ())),
                              preferred_element_type=f32) + 1.0
    dinv = jnp.where(deg > 0, jax.lax.rsqrt(deg), 0.0)

    def agg_exact(g):
        gh = g.astype(bf16)
        gl = (g - gh.astype(f32)).astype(bf16)
        dims = (((0,), (0,)), ((), ()))
        return (jax.lax.dot_general(adjb, gh, dims, preferred_element_type=f32)
                + jax.lax.dot_general(adjb, gl, dims,
                                      preferred_element_type=f32))

    def gcn_layer(h, W_ref, b_ref):
        g = jnp.dot(h, W_ref[...], preferred_element_type=f32,
                    precision=hi) * dinv
        return (agg_exact(g) + g) * dinv + b_ref[...]

    h1 = jax.nn.relu(gcn_layer(emb_ref[...], W1_ref, b1_ref))
    h2 = jax.nn.relu(gcn_layer(h1, W2_ref, b2_ref)) + h1
    h3 = jax.nn.relu(gcn_layer(h2, W3_ref, b3_ref)) + h2
    mu = jnp.dot(h3, Wmu_ref[...], preferred_element_type=f32,
                 precision=hi) + bmu_ref[...]
    lv = jnp.dot(h3, Wlv_ref[...], preferred_element_type=f32,
                 precision=hi) + blv_ref[...]
    mu_ref[...] = mu
    lv_ref[...] = lv
    muT = mu.T
    ut_ref[...] = jnp.dot(P1aT_ref[...], muT, preferred_element_type=f32,
                          precision=hi) + p1c_ref[...]
    vt_ref[...] = jnp.dot(P1bT_ref[...], muT, preferred_element_type=f32,
                          precision=hi)


def _chunk_rows(g, acol, bblk, P2T, p2c, P3r, p3):
    t = bblk.shape[1]
    cols = [acol[:, g * _G + r][:, None] + bblk for r in range(_G)]
    hh = jax.nn.relu(jnp.concatenate(cols, axis=1))
    h2 = jax.nn.relu(jnp.dot(P2T, hh, preferred_element_type=jnp.float32)
                     + p2c)
    logit = jnp.dot(P3r, h2, preferred_element_type=jnp.float32) + p3
    return jax.nn.sigmoid(logit).reshape(_G, t)


def _decoder_body(idx_ref, u_ref, v_ref, P2T_ref, p2c_ref, P3r_ref,
                  p3_ref, out_ref, s_ref):
    k = pl.program_id(0)
    m = pl.program_id(1)
    bi = idx_ref[0, k]
    bj = idx_ref[1, k]
    t = out_ref.shape[0]

    @pl.when(m == 0)
    def _():
        ublk, vblk = u_ref[...], v_ref[...]
        P2T, p2c = P2T_ref[...], p2c_ref[...]
        P3r, p3 = P3r_ref[...], p3_ref[0, 0]

        @pl.when(bi != bj)
        def _():
            for g in range(t // _G):
                s_ref[pl.ds(g * _G, _G), :] = _chunk_rows(
                    g, ublk, vblk, P2T, p2c, P3r, p3)

        @pl.when(bi == bj)
        def _():
            for g in range(t // _G):
                s_up = _chunk_rows(g, ublk, vblk, P2T, p2c, P3r, p3)
                s_lo = _chunk_rows(g, vblk, ublk, P2T, p2c, P3r, p3)
                r = g * _G + jax.lax.broadcasted_iota(jnp.int32, (_G, t), 0)
                c = jax.lax.broadcasted_iota(jnp.int32, (_G, t), 1)
                s_ref[pl.ds(g * _G, _G), :] = jnp.where(
                    r < c, s_up, jnp.where(r > c, s_lo, 0.0))

        out_ref[...] = s_ref[...]

    @pl.when(m == 1)
    def _():
        out_ref[...] = s_ref[...].T


def kernel(adj_matrix, emb, W1, b1, W2, b2, W3, b3, Wmu, bmu, Wlv, blv,
           P1, p1, P2, p2, P3, p3):
    n = adj_matrix.shape[0]
    L = Wmu.shape[1]
    H2 = P1.shape[1]
    f32 = jnp.float32

    mu, logvar, UT, VT = pl.pallas_call(
        _encoder_body,
        out_shape=(
            jax.ShapeDtypeStruct((n, L), f32),
            jax.ShapeDtypeStruct((n, L), f32),
            jax.ShapeDtypeStruct((H2, n), f32),
            jax.ShapeDtypeStruct((H2, n), f32),
        ),
    )(adj_matrix, emb, W1, b1.reshape(1, -1), W2, b2.reshape(1, -1), W3,
      b3.reshape(1, -1), Wmu, bmu.reshape(1, -1), Wlv, blv.reshape(1, -1),
      P1[:L].T, P1[L:].T, p1.reshape(-1, 1))

    nb = n // _TILE
    tri_i, tri_j = np.triu_indices(nb)
    tile_idx = jnp.asarray(np.stack([tri_i, tri_j]), dtype=jnp.int32)

    full = lambda shape: pl.BlockSpec(shape, lambda k, m, s: (0,) * len(shape))
    adj_pred = pl.pallas_call(
        _decoder_body,
        grid_spec=pltpu.PrefetchScalarGridSpec(
            num_scalar_prefetch=1,
            grid=(tri_i.size, 2),
            in_specs=[
                pl.BlockSpec((H2, _TILE), lambda k, m, s: (0, s[0, k])),
                pl.BlockSpec((H2, _TILE), lambda k, m, s: (0, s[1, k])),
                full((P2.shape[1], P2.shape[0])), full((P2.shape[1], 1)),
                full((1, P2.shape[1])), full((1, 1)),
            ],
            out_specs=pl.BlockSpec(
                (_TILE, _TILE),
                lambda k, m, s: (jnp.where(m == 0, s[0, k], s[1, k]),
                                 jnp.where(m == 0, s[1, k], s[0, k]))),
            scratch_shapes=[pltpu.VMEM((_TILE, _TILE), f32)],
        ),
        out_shape=jax.ShapeDtypeStruct((n, n), f32),
    )(tile_idx, UT, VT, P2.T, p2.reshape(-1, 1), P3.reshape(1, -1),
      p3.reshape(1, 1))

    return (adj_pred, mu, logvar, mu)


# speedup vs baseline: 659.8948x; 2.6548x over previous
import jax
import jax.numpy as jnp
import numpy as np
from jax.experimental import pallas as pl
from jax.experimental.pallas import tpu as pltpu

_TILE = 512
_G = 16


def _encode(adj_ref, emb_ref, W1_ref, b1_ref, W2_ref, b2_ref, W3_ref,
            b3_ref, Wmu_ref, bmu_ref, Wlv_ref, blv_ref, P1aT_ref,
            P1bT_ref, p1c_ref, mu_ref, lv_ref, ut_s, vt_s):
    f32 = jnp.float32
    bf16 = jnp.bfloat16
    hi = jax.lax.Precision.HIGHEST
    adj = adj_ref[...]
    n = adj.shape[0]
    adjb = adj.astype(bf16)
    ones = jnp.ones((n, 1), bf16)
    deg = jax.lax.dot_general(adjb, ones, (((0,), (0,)), ((), ())),
                              preferred_element_type=f32) + 1.0
    dinv = jnp.where(deg > 0, jax.lax.rsqrt(deg), 0.0)

    def agg_exact(g):
        gh = g.astype(bf16)
        gl = (g - gh.astype(f32)).astype(bf16)
        cat = jnp.concatenate([gh, gl], axis=1)
        out = jax.lax.dot_general(adjb, cat, (((0,), (0,)), ((), ())),
                                  preferred_element_type=f32)
        w = g.shape[1]
        return out[:, :w] + out[:, w:]

    def gcn_layer(h, W_ref, b_ref):
        g = jnp.dot(h, W_ref[...], preferred_element_type=f32,
                    precision=hi) * dinv
        return (agg_exact(g) + g) * dinv + b_ref[...]

    h1 = jax.nn.relu(gcn_layer(emb_ref[...], W1_ref, b1_ref))
    h2 = jax.nn.relu(gcn_layer(h1, W2_ref, b2_ref)) + h1
    h3 = jax.nn.relu(gcn_layer(h2, W3_ref, b3_ref)) + h2
    mu = jnp.dot(h3, Wmu_ref[...], preferred_element_type=f32,
                 precision=hi) + bmu_ref[...]
    lv = jnp.dot(h3, Wlv_ref[...], preferred_element_type=f32,
                 precision=hi) + blv_ref[...]
    mu_ref[...] = mu
    lv_ref[...] = lv
    muT = mu.T
    ut_s[...] = jnp.dot(P1aT_ref[...], muT, preferred_element_type=f32,
                        precision=hi) + p1c_ref[...]
    vt_s[...] = jnp.dot(P1bT_ref[...], muT, preferred_element_type=f32,
                        precision=hi)


def _chunk_rows(g, ublk, vblk, P2T, p2c, P3r, p3):
    t = vblk.shape[1]
    cols = [jnp.broadcast_to(ublk[:, g * _G + r:g * _G + r + 1],
                             vblk.shape) + vblk for r in range(_G)]
    hh = jax.nn.relu(jnp.concatenate(cols, axis=1))
    h2 = jax.nn.relu(jnp.dot(P2T, hh, preferred_element_type=jnp.float32)
                     + p2c)
    logit = jnp.dot(P3r, h2, preferred_element_type=jnp.float32) + p3
    return jax.nn.sigmoid(logit).reshape(_G, t)


def _fused_body(idx_ref, adj_ref, emb_ref, W1_ref, b1_ref, W2_ref, b2_ref,
                W3_ref, b3_ref, Wmu_ref, bmu_ref, Wlv_ref, blv_ref, P1aT_ref,
                P1bT_ref, p1c_ref, P2T_ref, p2c_ref, P3r_ref, p3_ref,
                mu_ref, lv_ref, out_ref, ut_s, vt_s, s_ref):
    k = pl.program_id(0)
    ph = idx_ref[0, k]
    bi = idx_ref[1, k]
    bj = idx_ref[2, k]
    t = out_ref.shape[0]

    @pl.when(ph == 0)
    def _():
        _encode(adj_ref, emb_ref, W1_ref, b1_ref, W2_ref, b2_ref, W3_ref,
                b3_ref, Wmu_ref, bmu_ref, Wlv_ref, blv_ref, P1aT_ref,
                P1bT_ref, p1c_ref, mu_ref, lv_ref, ut_s, vt_s)

    @pl.when(ph == 1)
    def _():
        ublk = ut_s[:, pl.ds(bi * t, t)]
        vblk = vt_s[:, pl.ds(bj * t, t)]
        P2T, p2c = P2T_ref[...], p2c_ref[...]
        P3r, p3 = P3r_ref[...], p3_ref[0, 0]
        for g in range(t // _G):
            s_ref[pl.ds(g * _G, _G), :] = _chunk_rows(
                g, ublk, vblk, P2T, p2c, P3r, p3)

        @pl.when(bi == bj)
        def _():
            s = s_ref[...]
            r = jax.lax.broadcasted_iota(jnp.int32, (t, t), 0)
            c = jax.lax.broadcasted_iota(jnp.int32, (t, t), 1)
            su = jnp.where(r < c, s, 0.0)
            s_ref[...] = su + su.T

        out_ref[...] = s_ref[...]

    @pl.when(ph == 2)
    def _():
        out_ref[...] = s_ref[...].T


def kernel(adj_matrix, emb, W1, b1, W2, b2, W3, b3, Wmu, bmu, Wlv, blv,
           P1, p1, P2, p2, P3, p3):
    n = adj_matrix.shape[0]
    L = Wmu.shape[1]
    H2 = P1.shape[1]
    f32 = jnp.float32

    nb = n // _TILE
    tri_i, tri_j = np.triu_indices(nb)
    steps = [[0, tri_i[0], tri_j[0]]]
    for i, j in zip(tri_i, tri_j):
        steps.append([1, i, j])
        if i != j:
            steps.append([2, j, i])
    idx = jnp.asarray(np.asarray(steps, dtype=np.int32).T)

    full = lambda shape: pl.BlockSpec(shape, lambda k, s: (0,) * len(shape))
    mu, logvar, adj_pred = pl.pallas_call(
        _fused_body,
        grid_spec=pltpu.PrefetchScalarGridSpec(
            num_scalar_prefetch=1,
            grid=(len(steps),),
            in_specs=[
                full((n, n)), full((n, emb.shape[1])),
                full(W1.shape), full((1, W1.shape[1])),
                full(W2.shape), full((1, W2.shape[1])),
                full(W3.shape), full((1, W3.shape[1])),
                full(Wmu.shape), full((1, L)),
                full(Wlv.shape), full((1, L)),
                full((H2, L)), full((H2, L)), full((H2, 1)),
                full((P2.shape[1], P2.shape[0])), full((P2.shape[1], 1)),
                full((1, P2.shape[1])), full((1, 1)),
            ],
            out_specs=[
                full((n, L)), full((n, L)),
                pl.BlockSpec((_TILE, _TILE),
                             lambda k, s: (s[1, k], s[2, k])),
            ],
            scratch_shapes=[pltpu.VMEM((H2, n), f32),
                            pltpu.VMEM((H2, n), f32),
                            pltpu.VMEM((_TILE, _TILE), f32)],
        ),
        out_shape=(
            jax.ShapeDtypeStruct((n, L), f32),
            jax.ShapeDtypeStruct((n, L), f32),
            jax.ShapeDtypeStruct((n, n), f32),
        ),
    )(idx, adj_matrix, emb, W1, b1.reshape(1, -1), W2, b2.reshape(1, -1),
      W3, b3.reshape(1, -1), Wmu, bmu.reshape(1, -1), Wlv, blv.reshape(1, -1),
      P1[:L].T, P1[L:].T, p1.reshape(-1, 1), P2.T, p2.reshape(-1, 1),
      P3.reshape(1, -1), p3.reshape(1, 1))

    return (adj_pred, mu, logvar, mu)


# speedup vs baseline: 761.3391x; 1.1537x over previous
import jax
import jax.numpy as jnp
import numpy as np
from jax.experimental import pallas as pl
from jax.experimental.pallas import tpu as pltpu

_TILE = 512
_G = 16


def _encode(adj_ref, emb_ref, W1_ref, b1_ref, W2_ref, b2_ref, W3_ref,
            b3_ref, Wmu_ref, bmu_ref, Wlv_ref, blv_ref, P1aT_ref,
            P1bT_ref, p1c_ref, mu_ref, lv_ref, ut_s, vt_s):
    f32 = jnp.float32
    bf16 = jnp.bfloat16
    hi = jax.lax.Precision.HIGHEST
    adj = adj_ref[...]
    n = adj.shape[0]
    adjb = adj.astype(bf16)
    ones = jnp.ones((n, 1), bf16)
    deg = jax.lax.dot_general(adjb, ones, (((0,), (0,)), ((), ())),
                              preferred_element_type=f32) + 1.0
    dinv = jnp.where(deg > 0, jax.lax.rsqrt(deg), 0.0)

    def agg_exact(g):
        gh = g.astype(bf16)
        gl = (g - gh.astype(f32)).astype(bf16)
        cat = jnp.concatenate([gh, gl], axis=1)
        out = jax.lax.dot_general(adjb, cat, (((0,), (0,)), ((), ())),
                                  preferred_element_type=f32)
        w = g.shape[1]
        return out[:, :w] + out[:, w:]

    def gcn_layer(h, W_ref, b_ref):
        g = jnp.dot(h, W_ref[...], preferred_element_type=f32,
                    precision=hi) * dinv
        return (agg_exact(g) + g) * dinv + b_ref[...]

    h1 = jax.nn.relu(gcn_layer(emb_ref[...], W1_ref, b1_ref))
    h2 = jax.nn.relu(gcn_layer(h1, W2_ref, b2_ref)) + h1
    h3 = jax.nn.relu(gcn_layer(h2, W3_ref, b3_ref)) + h2
    mu = jnp.dot(h3, Wmu_ref[...], preferred_element_type=f32,
                 precision=hi) + bmu_ref[...]
    lv = jnp.dot(h3, Wlv_ref[...], preferred_element_type=f32,
                 precision=hi) + blv_ref[...]
    mu_ref[...] = mu
    lv_ref[...] = lv
    muT = mu.T
    ut_s[...] = jnp.dot(P1aT_ref[...], muT, preferred_element_type=f32,
                        precision=hi) + p1c_ref[...]
    vt_s[...] = jnp.dot(P1bT_ref[...], muT, preferred_element_type=f32,
                        precision=hi)


def _chunk_rows(g, ublk, vblk, P2T, p2c, P3r, p3):
    t = vblk.shape[1]
    cols = [jnp.broadcast_to(ublk[:, g * _G + r:g * _G + r + 1],
                             vblk.shape) + vblk for r in range(_G)]
    hh = jax.nn.relu(jnp.concatenate(cols, axis=1))
    h2 = jax.nn.relu(jnp.dot(P2T, hh, preferred_element_type=jnp.float32)
                     + p2c)
    logit = jnp.dot(P3r, h2, preferred_element_type=jnp.float32) + p3
    return jax.nn.sigmoid(logit).reshape(_G, t)


def _fused_body(idx_ref, adj_ref, emb_ref, W1_ref, b1_ref, W2_ref, b2_ref,
                W3_ref, b3_ref, Wmu_ref, bmu_ref, Wlv_ref, blv_ref, P1aT_ref,
                P1bT_ref, p1c_ref, P2T_ref, p2c_ref, P3r_ref, p3_ref,
                mu_ref, lv_ref, out_ref, ut_s, vt_s, s_ref):
    k = pl.program_id(0)
    ph = idx_ref[0, k]
    bi = idx_ref[1, k]
    bj = idx_ref[2, k]
    t = out_ref.shape[0]

    @pl.when(ph == 0)
    def _():
        _encode(adj_ref, emb_ref, W1_ref, b1_ref, W2_ref, b2_ref, W3_ref,
                b3_ref, Wmu_ref, bmu_ref, Wlv_ref, blv_ref, P1aT_ref,
                P1bT_ref, p1c_ref, mu_ref, lv_ref, ut_s, vt_s)

    @pl.when(ph == 1)
    def _():
        bf16 = jnp.bfloat16
        ublk = ut_s[:, pl.ds(bi * t, t)].astype(bf16)
        vblk = vt_s[:, pl.ds(bj * t, t)].astype(bf16)
        P2T, p2c = P2T_ref[...].astype(bf16), p2c_ref[...]
        P3r, p3 = P3r_ref[...], p3_ref[0, 0]
        for g in range(t // _G):
            s_ref[pl.ds(g * _G, _G), :] = _chunk_rows(
                g, ublk, vblk, P2T, p2c, P3r, p3)

        @pl.when(bi == bj)
        def _():
            s = s_ref[...]
            r = jax.lax.broadcasted_iota(jnp.int32, (t, t), 0)
            c = jax.lax.broadcasted_iota(jnp.int32, (t, t), 1)
            su = jnp.where(r < c, s, 0.0)
            s_ref[...] = su + su.T

        out_ref[...] = s_ref[...]

    @pl.when(ph == 2)
    def _():
        out_ref[...] = s_ref[...].T


def kernel(adj_matrix, emb, W1, b1, W2, b2, W3, b3, Wmu, bmu, Wlv, blv,
           P1, p1, P2, p2, P3, p3):
    n = adj_matrix.shape[0]
    L = Wmu.shape[1]
    H2 = P1.shape[1]
    f32 = jnp.float32

    nb = n // _TILE
    tri_i, tri_j = np.triu_indices(nb)
    steps = [[0, tri_i[0], tri_j[0]]]
    for i, j in zip(tri_i, tri_j):
        steps.append([1, i, j])
        if i != j:
            steps.append([2, j, i])
    idx = jnp.asarray(np.asarray(steps, dtype=np.int32).T)

    full = lambda shape: pl.BlockSpec(shape, lambda k, s: (0,) * len(shape))
    mu, logvar, adj_pred = pl.pallas_call(
        _fused_body,
        grid_spec=pltpu.PrefetchScalarGridSpec(
            num_scalar_prefetch=1,
            grid=(len(steps),),
            in_specs=[
                full((n, n)), full((n, emb.shape[1])),
                full(W1.shape), full((1, W1.shape[1])),
                full(W2.shape), full((1, W2.shape[1])),
                full(W3.shape), full((1, W3.shape[1])),
                full(Wmu.shape), full((1, L)),
                full(Wlv.shape), full((1, L)),
                full((H2, L)), full((H2, L)), full((H2, 1)),
                full((P2.shape[1], P2.shape[0])), full((P2.shape[1], 1)),
                full((1, P2.shape[1])), full((1, 1)),
            ],
            out_specs=[
                full((n, L)), full((n, L)),
                pl.BlockSpec((_TILE, _TILE),
                             lambda k, s: (s[1, k], s[2, k])),
            ],
            scratch_shapes=[pltpu.VMEM((H2, n), f32),
                            pltpu.VMEM((H2, n), f32),
                            pltpu.VMEM((_TILE, _TILE), f32)],
        ),
        out_shape=(
            jax.ShapeDtypeStruct((n, L), f32),
            jax.ShapeDtypeStruct((n, L), f32),
            jax.ShapeDtypeStruct((n, n), f32),
        ),
    )(idx, adj_matrix, emb, W1, b1.reshape(1, -1), W2, b2.reshape(1, -1),
      W3, b3.reshape(1, -1), Wmu, bmu.reshape(1, -1), Wlv, blv.reshape(1, -1),
      P1[:L].T, P1[L:].T, p1.reshape(-1, 1), P2.T, p2.reshape(-1, 1),
      P3.reshape(1, -1), p3.reshape(1, 1))

    return (adj_pred, mu, logvar, mu)


# speedup vs baseline: 805.1727x; 1.0576x over previous
import jax
import jax.numpy as jnp
import numpy as np
from jax.experimental import pallas as pl
from jax.experimental.pallas import tpu as pltpu

_TILE = 512
_G = 16


def _encode(adj_ref, emb_ref, W1_ref, b1_ref, W2_ref, b2_ref, W3_ref,
            b3_ref, Wmu_ref, bmu_ref, Wlv_ref, blv_ref, P1aT_ref,
            P1bT_ref, p1c_ref, mu_ref, lv_ref, ut_s, vt_s):
    f32 = jnp.float32
    bf16 = jnp.bfloat16
    hi = jax.lax.Precision.HIGHEST
    adj = adj_ref[...]
    n = adj.shape[0]
    adjb = adj.astype(bf16)
    ones = jnp.ones((n, 1), bf16)
    deg = jax.lax.dot_general(adjb, ones, (((0,), (0,)), ((), ())),
                              preferred_element_type=f32) + 1.0
    dinv = jnp.where(deg > 0, jax.lax.rsqrt(deg), 0.0)

    def agg_exact(g):
        gh = g.astype(bf16)
        gl = (g - gh.astype(f32)).astype(bf16)
        cat = jnp.concatenate([gh, gl], axis=1)
        out = jax.lax.dot_general(adjb, cat, (((0,), (0,)), ((), ())),
                                  preferred_element_type=f32)
        w = g.shape[1]
        return out[:, :w] + out[:, w:]

    def gcn_layer(h, W_ref, b_ref):
        g = jnp.dot(h, W_ref[...], preferred_element_type=f32,
                    precision=hi) * dinv
        return (agg_exact(g) + g) * dinv + b_ref[...]

    h1 = jax.nn.relu(gcn_layer(emb_ref[...], W1_ref, b1_ref))
    h2 = jax.nn.relu(gcn_layer(h1, W2_ref, b2_ref)) + h1
    h3 = jax.nn.relu(gcn_layer(h2, W3_ref, b3_ref)) + h2
    mu = jnp.dot(h3, Wmu_ref[...], preferred_element_type=f32,
                 precision=hi) + bmu_ref[...]
    lv = jnp.dot(h3, Wlv_ref[...], preferred_element_type=f32,
                 precision=hi) + blv_ref[...]
    mu_ref[...] = mu
    lv_ref[...] = lv
    muT = mu.T
    ut_s[...] = jnp.dot(P1aT_ref[...], muT, preferred_element_type=f32,
                        precision=hi) + p1c_ref[...]
    vt_s[...] = jnp.dot(P1bT_ref[...], muT, preferred_element_type=f32,
                        precision=hi)


def _chunk_rows(g, ublk, vblk, P2T, p2c, P3r, p3):
    t = vblk.shape[1]
    cols = [jnp.broadcast_to(ublk[:, g * _G + r:g * _G + r + 1],
                             vblk.shape) + vblk for r in range(_G)]
    hh = jax.nn.relu(jnp.concatenate(cols, axis=1))
    h2 = jax.nn.relu(jnp.dot(P2T, hh, preferred_element_type=jnp.float32)
                     + p2c)
    logit = jnp.dot(P3r, h2, preferred_element_type=jnp.float32) + p3
    return jax.nn.sigmoid(logit).reshape(_G, t)


def _fused_body(idx_ref, adj_ref, emb_ref, W1_ref, b1_ref, W2_ref, b2_ref,
                W3_ref, b3_ref, Wmu_ref, bmu_ref, Wlv_ref, blv_ref, P1aT_ref,
                P1bT_ref, p1c_ref, P2T_ref, p2c_ref, P3r_ref, p3_ref,
                mu_ref, lv_ref, out_ref, ut_s, vt_s, s_ref):
    k = pl.program_id(0)
    ph = idx_ref[0, k]
    bi = idx_ref[1, k]
    bj = idx_ref[2, k]
    t = out_ref.shape[0]

    @pl.when(ph == 0)
    def _():
        _encode(adj_ref, emb_ref, W1_ref, b1_ref, W2_ref, b2_ref, W3_ref,
                b3_ref, Wmu_ref, bmu_ref, Wlv_ref, blv_ref, P1aT_ref,
                P1bT_ref, p1c_ref, mu_ref, lv_ref, ut_s, vt_s)

    @pl.when(ph == 1)
    def _():
        bf16 = jnp.bfloat16
        ublk = ut_s[:, pl.ds(bi * t, t)].astype(bf16)
        vblk = vt_s[:, pl.ds(bj * t, t)].astype(bf16)
        P2T, p2c = P2T_ref[...].astype(bf16), p2c_ref[...]
        P3r, p3 = P3r_ref[...], p3_ref[0, 0]

        @pl.when(bi != bj)
        def _():
            for g in range(t // _G):
                s_ref[pl.ds(g * _G, _G), :] = _chunk_rows(
                    g, ublk, vblk, P2T, p2c, P3r, p3)

        @pl.when(bi == bj)
        def _():
            for g in range(t // _G):
                c0 = (g * _G // 128) * 128
                s_ref[pl.ds(g * _G, _G), c0:] = _chunk_rows(
                    g, ublk, vblk[:, c0:], P2T, p2c, P3r, p3)
            s = s_ref[...]
            r = jax.lax.broadcasted_iota(jnp.int32, (t, t), 0)
            c = jax.lax.broadcasted_iota(jnp.int32, (t, t), 1)
            su = jnp.where(r < c, s, 0.0)
            s_ref[...] = su + su.T

        out_ref[...] = s_ref[...]

    @pl.when(ph == 2)
    def _():
        out_ref[...] = s_ref[...].T


def kernel(adj_matrix, emb, W1, b1, W2, b2, W3, b3, Wmu, bmu, Wlv, blv,
           P1, p1, P2, p2, P3, p3):
    n = adj_matrix.shape[0]
    L = Wmu.shape[1]
    H2 = P1.shape[1]
    f32 = jnp.float32

    nb = n // _TILE
    tri_i, tri_j = np.triu_indices(nb)
    steps = [[0, tri_i[0], tri_j[0]]]
    for i, j in zip(tri_i, tri_j):
        steps.append([1, i, j])
        if i != j:
            steps.append([2, j, i])
    idx = jnp.asarray(np.asarray(steps, dtype=np.int32).T)

    full = lambda shape: pl.BlockSpec(shape, lambda k, s: (0,) * len(shape))
    mu, logvar, adj_pred = pl.pallas_call(
        _fused_body,
        grid_spec=pltpu.PrefetchScalarGridSpec(
            num_scalar_prefetch=1,
            grid=(len(steps),),
            in_specs=[
                full((n, n)), full((n, emb.shape[1])),
                full(W1.shape), full((1, W1.shape[1])),
                full(W2.shape), full((1, W2.shape[1])),
                full(W3.shape), full((1, W3.shape[1])),
                full(Wmu.shape), full((1, L)),
                full(Wlv.shape), full((1, L)),
                full((H2, L)), full((H2, L)), full((H2, 1)),
                full((P2.shape[1], P2.shape[0])), full((P2.shape[1], 1)),
                full((1, P2.shape[1])), full((1, 1)),
            ],
            out_specs=[
                full((n, L)), full((n, L)),
                pl.BlockSpec((_TILE, _TILE),
                             lambda k, s: (s[1, k], s[2, k])),
            ],
            scratch_shapes=[pltpu.VMEM((H2, n), f32),
                            pltpu.VMEM((H2, n), f32),
                            pltpu.VMEM((_TILE, _TILE), f32)],
        ),
        out_shape=(
            jax.ShapeDtypeStruct((n, L), f32),
            jax.ShapeDtypeStruct((n, L), f32),
            jax.ShapeDtypeStruct((n, n), f32),
        ),
    )(idx, adj_matrix, emb, W1, b1.reshape(1, -1), W2, b2.reshape(1, -1),
      W3, b3.reshape(1, -1), Wmu, bmu.reshape(1, -1), Wlv, blv.reshape(1, -1),
      P1[:L].T, P1[L:].T, p1.reshape(-1, 1), P2.T, p2.reshape(-1, 1),
      P3.reshape(1, -1), p3.reshape(1, 1))

    return (adj_pred, mu, logvar, mu)


# speedup vs baseline: 878.3558x; 1.0909x over previous
import jax
import jax.numpy as jnp
import numpy as np
from jax.experimental import pallas as pl
from jax.experimental.pallas import tpu as pltpu

_TILE = 512
_G = 32


def _encode(adj_ref, emb_ref, W1_ref, b1_ref, W2_ref, b2_ref, W3_ref,
            b3_ref, Wmu_ref, bmu_ref, Wlv_ref, blv_ref, P1aT_ref,
            P1bT_ref, p1c_ref, mu_ref, lv_ref, ut_s, vt_s):
    f32 = jnp.float32
    bf16 = jnp.bfloat16
    hi = jax.lax.Precision.HIGHEST
    adj = adj_ref[...]
    n = adj.shape[0]
    adjb = adj.astype(bf16)
    ones = jnp.ones((n, 1), bf16)
    deg = jax.lax.dot_general(adjb, ones, (((0,), (0,)), ((), ())),
                              preferred_element_type=f32) + 1.0
    dinv = jnp.where(deg > 0, jax.lax.rsqrt(deg), 0.0)

    def agg_exact(g):
        gh = g.astype(bf16)
        gl = (g - gh.astype(f32)).astype(bf16)
        cat = jnp.concatenate([gh, gl], axis=1)
        out = jax.lax.dot_general(adjb, cat, (((0,), (0,)), ((), ())),
                                  preferred_element_type=f32)
        w = g.shape[1]
        return out[:, :w] + out[:, w:]

    def gcn_layer(h, W_ref, b_ref):
        g = jnp.dot(h, W_ref[...], preferred_element_type=f32,
                    precision=hi) * dinv
        return (agg_exact(g) + g) * dinv + b_ref[...]

    h1 = jax.nn.relu(gcn_layer(emb_ref[...], W1_ref, b1_ref))
    h2 = jax.nn.relu(gcn_layer(h1, W2_ref, b2_ref)) + h1
    h3 = jax.nn.relu(gcn_layer(h2, W3_ref, b3_ref)) + h2
    mu = jnp.dot(h3, Wmu_ref[...], preferred_element_type=f32,
                 precision=hi) + bmu_ref[...]
    lv = jnp.dot(h3, Wlv_ref[...], preferred_element_type=f32,
                 precision=hi) + blv_ref[...]
    mu_ref[...] = mu
    lv_ref[...] = lv
    muT = mu.T
    ut_s[...] = jnp.dot(P1aT_ref[...], muT, preferred_element_type=f32,
                        precision=hi) + p1c_ref[...]
    vt_s[...] = jnp.dot(P1bT_ref[...], muT, preferred_element_type=f32,
                        precision=hi)


def _chunk_rows(g, ublk, vblk, P2T, p2c, P3r, p3):
    t = vblk.shape[1]
    cols = [jnp.broadcast_to(ublk[:, g * _G + r:g * _G + r + 1],
                             vblk.shape) + vblk for r in range(_G)]
    hh = jax.nn.relu(jnp.concatenate(cols, axis=1))
    h2 = jax.nn.relu(jnp.dot(P2T, hh, preferred_element_type=jnp.float32)
                     + p2c)
    logit = jnp.dot(P3r, h2, preferred_element_type=jnp.float32) + p3
    return jax.nn.sigmoid(logit).reshape(_G, t)


def _fused_body(idx_ref, adj_ref, emb_ref, W1_ref, b1_ref, W2_ref, b2_ref,
                W3_ref, b3_ref, Wmu_ref, bmu_ref, Wlv_ref, blv_ref, P1aT_ref,
                P1bT_ref, p1c_ref, P2T_ref, p2c_ref, P3r_ref, p3_ref,
                mu_ref, lv_ref, out_ref, ut_s, vt_s, s_ref):
    k = pl.program_id(0)
    ph = idx_ref[0, k]
    bi = idx_ref[1, k]
    bj = idx_ref[2, k]
    t = out_ref.shape[0]

    @pl.when(ph == 0)
    def _():
        _encode(adj_ref, emb_ref, W1_ref, b1_ref, W2_ref, b2_ref, W3_ref,
                b3_ref, Wmu_ref, bmu_ref, Wlv_ref, blv_ref, P1aT_ref,
                P1bT_ref, p1c_ref, mu_ref, lv_ref, ut_s, vt_s)

    @pl.when(ph == 1)
    def _():
        bf16 = jnp.bfloat16
        ublk = ut_s[:, pl.ds(bi * t, t)].astype(bf16)
        vblk = vt_s[:, pl.ds(bj * t, t)].astype(bf16)
        P2T, p2c = P2T_ref[...].astype(bf16), p2c_ref[...]
        P3r, p3 = P3r_ref[...], p3_ref[0, 0]

        @pl.when(bi != bj)
        def _():
            for g in range(t // _G):
                s_ref[pl.ds(g * _G, _G), :] = _chunk_rows(
                    g, ublk, vblk, P2T, p2c, P3r, p3)

        @pl.when(bi == bj)
        def _():
            for g in range(t // _G):
                c0 = (g * _G // 128) * 128
                s_ref[pl.ds(g * _G, _G), c0:] = _chunk_rows(
                    g, ublk, vblk[:, c0:], P2T, p2c, P3r, p3)
            s = s_ref[...]
            r = jax.lax.broadcasted_iota(jnp.int32, (t, t), 0)
            c = jax.lax.broadcasted_iota(jnp.int32, (t, t), 1)
            su = jnp.where(r < c, s, 0.0)
            s_ref[...] = su + su.T

        out_ref[...] = s_ref[...]

    @pl.when(ph == 2)
    def _():
        out_ref[...] = s_ref[...].T


def kernel(adj_matrix, emb, W1, b1, W2, b2, W3, b3, Wmu, bmu, Wlv, blv,
           P1, p1, P2, p2, P3, p3):
    n = adj_matrix.shape[0]
    L = Wmu.shape[1]
    H2 = P1.shape[1]
    f32 = jnp.float32

    nb = n // _TILE
    tri_i, tri_j = np.triu_indices(nb)
    steps = [[0, tri_i[0], tri_j[0]]]
    for i, j in zip(tri_i, tri_j):
        steps.append([1, i, j])
        if i != j:
            steps.append([2, j, i])
    idx = jnp.asarray(np.asarray(steps, dtype=np.int32).T)

    full = lambda shape: pl.BlockSpec(shape, lambda k, s: (0,) * len(shape))
    mu, logvar, adj_pred = pl.pallas_call(
        _fused_body,
        grid_spec=pltpu.PrefetchScalarGridSpec(
            num_scalar_prefetch=1,
            grid=(len(steps),),
            in_specs=[
                full((n, n)), full((n, emb.shape[1])),
                full(W1.shape), full((1, W1.shape[1])),
                full(W2.shape), full((1, W2.shape[1])),
                full(W3.shape), full((1, W3.shape[1])),
                full(Wmu.shape), full((1, L)),
                full(Wlv.shape), full((1, L)),
                full((H2, L)), full((H2, L)), full((H2, 1)),
                full((P2.shape[1], P2.shape[0])), full((P2.shape[1], 1)),
                full((1, P2.shape[1])), full((1, 1)),
            ],
            out_specs=[
                full((n, L)), full((n, L)),
                pl.BlockSpec((_TILE, _TILE),
                             lambda k, s: (s[1, k], s[2, k])),
            ],
            scratch_shapes=[pltpu.VMEM((H2, n), f32),
                            pltpu.VMEM((H2, n), f32),
                            pltpu.VMEM((_TILE, _TILE), f32)],
        ),
        out_shape=(
            jax.ShapeDtypeStruct((n, L), f32),
            jax.ShapeDtypeStruct((n, L), f32),
            jax.ShapeDtypeStruct((n, n), f32),
        ),
    )(idx, adj_matrix, emb, W1, b1.reshape(1, -1), W2, b2.reshape(1, -1),
      W3, b3.reshape(1, -1), Wmu, bmu.reshape(1, -1), Wlv, blv.reshape(1, -1),
      P1[:L].T, P1[L:].T, p1.reshape(-1, 1), P2.T, p2.reshape(-1, 1),
      P3.reshape(1, -1), p3.reshape(1, 1))

    return (adj_pred, mu, logvar, mu)
